# baseline (speedup 1.0000x reference)
import jax
import jax.numpy as jnp
from jax.experimental import pallas as pl
from jax.experimental.pallas import tpu as pltpu
from jax.experimental.pallas import tpu_sc as plsc

ROWS = 100000
HIDDEN = 256
EMBED = 128
BATCH = 4096
HIST = 20
N_TOK = BATCH * HIST

CHUNK = 1000
NCHUNK = ROWS // CHUNK
NBUF = 10


def _proj_body(enc_hbm, w_ref, b_ref, p_hbm, in_buf, out_buf, in_sems, out_sems):
    g = pl.program_id(0)
    slot = jax.lax.rem(g, NBUF)

    def in_copy(chunk, slot):
        return pltpu.make_async_copy(
            enc_hbm.at[pl.ds(chunk * CHUNK, CHUNK)],
            in_buf.at[slot],
            in_sems.at[slot],
        )

    def out_copy(chunk, slot):
        return pltpu.make_async_copy(
            out_buf.at[slot],
            p_hbm.at[pl.ds(chunk * CHUNK, CHUNK)],
            out_sems.at[slot],
        )

    @pl.when(g == 0)
    def _prologue():
        for k in range(NBUF):
            in_copy(k, k).start()

    in_copy(g, slot).wait()

    @pl.when(g >= NBUF)
    def _wait_out():
        out_copy(g - NBUF, slot).wait()

    out_buf[slot] = jnp.dot(
        in_buf[slot], w_ref[...],
        preferred_element_type=jnp.float32,
        precision=jax.lax.Precision.DEFAULT,
    ) + b_ref[...]

    out_copy(g, slot).start()

    @pl.when(g + NBUF < NCHUNK)
    def _next_in():
        in_copy(g + NBUF, slot).start()

    @pl.when(g == NCHUNK - 1)
    def _drain():
        for k in range(NBUF):
            out_copy(g - NBUF + 1 + k, jax.lax.rem(g + 1 + k, NBUF)).wait()


def _project_table(enc, W, b):
    return pl.pallas_call(
        _proj_body,
        grid=(NCHUNK,),
        in_specs=[
            pl.BlockSpec(memory_space=pltpu.MemorySpace.HBM),
            pl.BlockSpec((HIDDEN, EMBED), lambda i: (0, 0)),
            pl.BlockSpec((1, EMBED), lambda i: (0, 0)),
        ],
        out_specs=pl.BlockSpec(memory_space=pltpu.MemorySpace.HBM),
        out_shape=jax.ShapeDtypeStruct((ROWS, EMBED), jnp.float32),
        scratch_shapes=[
            pltpu.VMEM((NBUF, CHUNK, HIDDEN), jnp.float32),
            pltpu.VMEM((NBUF, CHUNK, EMBED), jnp.float32),
            pltpu.SemaphoreType.DMA((NBUF,)),
            pltpu.SemaphoreType.DMA((NBUF,)),
        ],
    )(enc, W, b.reshape(1, EMBED))


BATCH_BLK = 16


def _sc_gather(table, idx):
    mesh = plsc.VectorSubcoreMesh(core_axis_name="core", subcore_axis_name="subcore")

    @pl.kernel(out_type=jax.ShapeDtypeStruct((BATCH, HIST, EMBED), jnp.float32),
               mesh=mesh,
               scratch_types=[pltpu.SemaphoreType.DMA])
    def k(tab_hbm, i_hbm, o_hbm, sem):
        def body(i_vmem, o_vmem):
            copies = [
                pltpu.async_copy(tab_hbm.at[i_vmem.at[p]], o_vmem.at[p], sem)
                for p in range(BATCH_BLK)
            ]
            for c in copies:
                c.wait()

        pltpu.emit_pipeline(
            body,
            grid=(BATCH // BATCH_BLK,),
            in_specs=[pl.BlockSpec((BATCH_BLK, HIST), index_map=lambda i: (i, 0))],
            out_specs=[pl.BlockSpec((BATCH_BLK, HIST, EMBED),
                                    index_map=lambda i: (i, 0, 0))],
            core_axis_name=("core", "subcore"),
            dimension_semantics=(pltpu.PARALLEL,),
        )(i_hbm, o_hbm)

    return k(table, idx)


def kernel(num, encodings, W, b):
    P = _project_table(encodings, W, b)
    idx = num.astype(jnp.int32)
    return _sc_gather(P, idx)
